# Initial kernel scaffold; baseline (speedup 1.0000x reference)
import jax
import jax.numpy as jnp
from jax.experimental import pallas as pl


def kernel(x, edge_index, W_enc, b_enc, W1, b1, W2, b2, W_out, b_out):
    raise NotImplementedError("write your pallas kernel here")


# speedup vs baseline: 8.2488x; 8.2488x over previous
import functools

import jax
import jax.numpy as jnp
from jax import lax
from jax.experimental import pallas as pl
from jax.experimental.pallas import tpu as pltpu
from jax.experimental.pallas import tpu_sc as plsc

N = 10000
NPAD = 10240
E = 320000
D = 128
H = 128
C = 40

NC = 2
NS = 16
NW = NC * NS
EPW = 10240
EPAD = NW * EPW
K = 128
NCHUNK = EPW // K
RPT = NPAD // NS

_mesh = plsc.VectorSubcoreMesh(core_axis_name="c", subcore_axis_name="s")


@functools.partial(
    pl.kernel,
    out_type=jax.ShapeDtypeStruct((NC * NPAD,), jnp.float32),
    mesh=_mesh,
    scratch_types=[
        pltpu.VMEM_SHARED((NPAD,), jnp.float32),
        pltpu.VMEM((2, 1, K), jnp.int32),
        pltpu.VMEM((K,), jnp.float32),
    ],
)
def _sc_degree(dst_hbm, zeros_hbm, out_hbm, deg_sh, ring, ones_v):
    core = lax.axis_index("c")
    sub = lax.axis_index("s")
    wid = sub * NC + core
    row0 = sub * RPT
    for r in range(K // 16):
        ones_v[pl.ds(16 * r, 16)] = jnp.ones((16,), jnp.float32)
    pltpu.sync_copy(zeros_hbm.at[pl.ds(row0, RPT)], deg_sh.at[pl.ds(row0, RPT)])
    plsc.subcore_barrier()

    pltpu.sync_copy(dst_hbm.at[wid, 0], ring.at[0])

    @pl.loop(0, NCHUNK, step=2)
    def _(j):
        for b in range(2):
            c = j + b

            @pl.when(c + 1 < NCHUNK)
            def _():
                pltpu.sync_copy(dst_hbm.at[wid, c + 1], ring.at[1 - b])

            pltpu.sync_copy(ones_v, deg_sh.at[ring.at[b, 0]], add=True)

    plsc.subcore_barrier()
    pltpu.sync_copy(deg_sh.at[pl.ds(row0, RPT)],
                    out_hbm.at[pl.ds(core * NPAD + row0, RPT)])


@functools.partial(
    pl.kernel,
    out_type=jax.ShapeDtypeStruct((NC, NPAD, H), jnp.float32),
    mesh=_mesh,
    scratch_types=[
        pltpu.VMEM_SHARED((NPAD, H), jnp.float32),
        pltpu.VMEM((2, 1, K), jnp.int32),
        pltpu.VMEM((2, 1, K), jnp.int32),
        pltpu.VMEM((2, K, H), jnp.float32),
        pltpu.SemaphoreType.DMA,
        pltpu.SemaphoreType.DMA,
    ],
)
def _sc_scatter(hp_hbm, src_hbm, dst_hbm, zeros_hbm, out_hbm,
                agg_sh, ring_s, ring_d, rows_v, gsem0, gsem1):
    core = lax.axis_index("c")
    sub = lax.axis_index("s")
    wid = sub * NC + core
    row0 = sub * RPT
    pltpu.sync_copy(zeros_hbm.at[pl.ds(row0, RPT)], agg_sh.at[pl.ds(row0, RPT)])
    plsc.subcore_barrier()

    pltpu.sync_copy(src_hbm.at[wid, 0], ring_s.at[0])
    pltpu.sync_copy(dst_hbm.at[wid, 0], ring_d.at[0])
    pltpu.async_copy(hp_hbm.at[ring_s.at[0, 0]], rows_v.at[0], gsem0)

    @pl.loop(0, NCHUNK, step=2)
    def _(j):
        for b in range(2):
            c = j + b
            sem_b = gsem0 if b == 0 else gsem1
            sem_o = gsem1 if b == 0 else gsem0

            @pl.when(c + 1 < NCHUNK)
            def _():
                pltpu.sync_copy(src_hbm.at[wid, c + 1], ring_s.at[1 - b])
                pltpu.sync_copy(dst_hbm.at[wid, c + 1], ring_d.at[1 - b])
                pltpu.async_copy(hp_hbm.at[ring_s.at[1 - b, 0]],
                                 rows_v.at[1 - b], sem_o)

            pltpu.make_async_copy(hp_hbm.at[ring_s.at[b, 0]],
                                  rows_v.at[b], sem_b).wait()
            pltpu.sync_copy(rows_v.at[b], agg_sh.at[ring_d.at[b, 0]], add=True)

    plsc.subcore_barrier()
    pltpu.sync_copy(agg_sh.at[pl.ds(row0, RPT)],
                    out_hbm.at[core, pl.ds(row0, RPT)])


R = 1024
G = NPAD // R

_row = lambda i: (i, 0)
_fix = lambda i: (0, 0)
_CT = (((1,), (1,)), ((), ()))


def _dis(d0_ref, d1_ref):
    return lax.rsqrt(1.0 + d0_ref[...] + d1_ref[...])


def _tc_prep_body(x_ref, w_ref, b_ref, d0_ref, d1_ref, out_ref):
    h = lax.dot_general(x_ref[...], w_ref[...], _CT,
                        preferred_element_type=jnp.float32) + b_ref[...]
    out_ref[...] = _dis(d0_ref, d1_ref) * h


def _tc_layer_body(s0_ref, s1_ref, hp_ref, d0_ref, d1_ref, w_ref, b_ref,
                   out_ref):
    dis = _dis(d0_ref, d1_ref)
    t = dis * (s0_ref[...] + s1_ref[...] + hp_ref[...])
    h = lax.dot_general(t, w_ref[...], _CT,
                        preferred_element_type=jnp.float32) + b_ref[...]
    out_ref[...] = dis * jnp.maximum(h, 0.0)


def _tc_final_body(s0_ref, s1_ref, hp_ref, d0_ref, d1_ref, w_ref, b_ref,
                   wo_ref, bo_ref, out_ref):
    dis = _dis(d0_ref, d1_ref)
    t = dis * (s0_ref[...] + s1_ref[...] + hp_ref[...])
    h = lax.dot_general(t, w_ref[...], _CT,
                        preferred_element_type=jnp.float32) + b_ref[...]
    h = jnp.maximum(h, 0.0)
    out_ref[...] = lax.dot_general(h, wo_ref[...], _CT,
                                   preferred_element_type=jnp.float32) + bo_ref[...]


_deg_spec = pl.BlockSpec((R, 1), _row)
_feat_spec = pl.BlockSpec((R, H), _row)
_w_spec = pl.BlockSpec((H, H), _fix)
_b_spec = pl.BlockSpec((1, H), _fix)

_tc_prep = pl.pallas_call(
    _tc_prep_body,
    grid=(G,),
    in_specs=[pl.BlockSpec((R, D), _row), pl.BlockSpec((H, D), _fix),
              _b_spec, _deg_spec, _deg_spec],
    out_specs=_feat_spec,
    out_shape=jax.ShapeDtypeStruct((NPAD, H), jnp.float32),
)

_tc_layer = pl.pallas_call(
    _tc_layer_body,
    grid=(G,),
    in_specs=[_feat_spec, _feat_spec, _feat_spec, _deg_spec, _deg_spec,
              _w_spec, _b_spec],
    out_specs=_feat_spec,
    out_shape=jax.ShapeDtypeStruct((NPAD, H), jnp.float32),
)

_tc_final = pl.pallas_call(
    _tc_final_body,
    grid=(G,),
    in_specs=[_feat_spec, _feat_spec, _feat_spec, _deg_spec, _deg_spec,
              _w_spec, _b_spec, _w_spec, _b_spec],
    out_specs=_feat_spec,
    out_shape=jax.ShapeDtypeStruct((NPAD, H), jnp.float32),
)


@jax.jit
def kernel(x, edge_index, W_enc, b_enc, W1, b1, W2, b2, W_out, b_out):
    npd = EPAD - E
    srcp = jnp.concatenate([edge_index[0], jnp.zeros((npd,), jnp.int32)])
    dstp = jnp.concatenate(
        [edge_index[1], N + (jnp.arange(npd, dtype=jnp.int32) % (NPAD - N))])
    src_i = srcp.reshape(NW, NCHUNK, 1, K)
    dst_i = dstp.reshape(NW, NCHUNK, 1, K)
    zeros_deg = jnp.zeros((NPAD,), jnp.float32)
    zeros_h = jnp.zeros((NPAD, H), jnp.float32)
    xp = jnp.zeros((NPAD, D), jnp.float32).at[:N].set(x)

    deg = _sc_degree(dst_i, zeros_deg)
    d0 = deg[:NPAD].reshape(NPAD, 1)
    d1 = deg[NPAD:].reshape(NPAD, 1)

    hp0 = _tc_prep(xp, W_enc, b_enc.reshape(1, H), d0, d1)
    s = _sc_scatter(hp0, src_i, dst_i, zeros_h)
    hp1 = _tc_layer(s[0], s[1], hp0, d0, d1, W1, b1.reshape(1, H))
    s2 = _sc_scatter(hp1, src_i, dst_i, zeros_h)

    wo = jnp.zeros((H, H), jnp.float32).at[:C].set(W_out)
    bo = jnp.zeros((1, H), jnp.float32).at[0, :C].set(b_out)
    out = _tc_final(s2[0], s2[1], hp1, d0, d1, W2, b2.reshape(1, H), wo, bo)
    return out[:N, :C]


# speedup vs baseline: 9.7680x; 1.1842x over previous
import functools

import jax
import jax.numpy as jnp
from jax import lax
from jax.experimental import pallas as pl
from jax.experimental.pallas import tpu as pltpu
from jax.experimental.pallas import tpu_sc as plsc

N = 10000
NPAD = 10240
E = 320000
D = 128
H = 128
C = 40

NC = 2
NS = 16
NW = NC * NS
EPW = 10240
EPAD = NW * EPW
K = 128
NCHUNK = EPW // K
RPT = NPAD // NS

_mesh = plsc.VectorSubcoreMesh(core_axis_name="c", subcore_axis_name="s")


@functools.partial(
    pl.kernel,
    out_type=jax.ShapeDtypeStruct((NC * NPAD,), jnp.float32),
    mesh=_mesh,
    scratch_types=[
        pltpu.VMEM_SHARED((NPAD,), jnp.float32),
        pltpu.VMEM((2, 1, K), jnp.int32),
        pltpu.VMEM((K,), jnp.float32),
    ],
)
def _sc_degree(dst_hbm, zeros_hbm, out_hbm, deg_sh, ring, ones_v):
    core = lax.axis_index("c")
    sub = lax.axis_index("s")
    wid = sub * NC + core
    row0 = sub * RPT
    for r in range(K // 16):
        ones_v[pl.ds(16 * r, 16)] = jnp.ones((16,), jnp.float32)
    pltpu.sync_copy(zeros_hbm.at[pl.ds(row0, RPT)], deg_sh.at[pl.ds(row0, RPT)])
    plsc.subcore_barrier()

    pltpu.sync_copy(dst_hbm.at[wid, 0], ring.at[0])

    @pl.loop(0, NCHUNK, step=2)
    def _(j):
        for b in range(2):
            c = j + b

            @pl.when(c + 1 < NCHUNK)
            def _():
                pltpu.sync_copy(dst_hbm.at[wid, c + 1], ring.at[1 - b])

            pltpu.sync_copy(ones_v, deg_sh.at[ring.at[b, 0]], add=True)

    plsc.subcore_barrier()
    pltpu.sync_copy(deg_sh.at[pl.ds(row0, RPT)],
                    out_hbm.at[pl.ds(core * NPAD + row0, RPT)])


@functools.partial(
    pl.kernel,
    out_type=jax.ShapeDtypeStruct((NC, NPAD, H), jnp.float32),
    mesh=_mesh,
    scratch_types=[
        pltpu.VMEM_SHARED((NPAD, H), jnp.float32),
        pltpu.VMEM((2, 1, K), jnp.int32),
        pltpu.VMEM((2, 1, K), jnp.int32),
        pltpu.VMEM((2, K, H), jnp.float32),
        pltpu.SemaphoreType.DMA,
        pltpu.SemaphoreType.DMA,
    ],
)
def _sc_scatter(hp_hbm, src_hbm, dst_hbm, zeros_hbm, out_hbm,
                agg_sh, ring_s, ring_d, rows_v, gsem0, gsem1):
    core = lax.axis_index("c")
    sub = lax.axis_index("s")
    wid = sub * NC + core
    row0 = sub * RPT
    pltpu.sync_copy(zeros_hbm.at[pl.ds(row0, RPT)], agg_sh.at[pl.ds(row0, RPT)])
    plsc.subcore_barrier()

    pltpu.sync_copy(src_hbm.at[wid, 0], ring_s.at[0])
    pltpu.sync_copy(dst_hbm.at[wid, 0], ring_d.at[0])
    pltpu.async_copy(hp_hbm.at[ring_s.at[0, 0]], rows_v.at[0], gsem0)

    @pl.loop(0, NCHUNK, step=2)
    def _(j):
        for b in range(2):
            c = j + b
            sem_b = gsem0 if b == 0 else gsem1
            sem_o = gsem1 if b == 0 else gsem0

            @pl.when(c + 1 < NCHUNK)
            def _():
                pltpu.sync_copy(src_hbm.at[wid, c + 1], ring_s.at[1 - b])
                pltpu.sync_copy(dst_hbm.at[wid, c + 1], ring_d.at[1 - b])
                pltpu.async_copy(hp_hbm.at[ring_s.at[1 - b, 0]],
                                 rows_v.at[1 - b], sem_o)

            pltpu.make_async_copy(hp_hbm.at[ring_s.at[b, 0]],
                                  rows_v.at[b], sem_b).wait()
            pltpu.sync_copy(rows_v.at[b], agg_sh.at[ring_d.at[b, 0]], add=True)

    plsc.subcore_barrier()
    pltpu.sync_copy(agg_sh.at[pl.ds(row0, RPT)],
                    out_hbm.at[core, pl.ds(row0, RPT)])


R = 1024
G = NPAD // R

_row = lambda i: (i, 0)
_fix = lambda i: (0, 0)
_CT = (((1,), (1,)), ((), ()))


def _dis(d0_ref, d1_ref):
    return lax.rsqrt(1.0 + d0_ref[...] + d1_ref[...])


def _tc_prep_body(x_ref, w_ref, b_ref, d0_ref, d1_ref, out_ref):
    h = lax.dot_general(x_ref[...], w_ref[...], _CT,
                        preferred_element_type=jnp.float32) + b_ref[...]
    out_ref[...] = _dis(d0_ref, d1_ref) * h


def _tc_layer_body(s0_ref, s1_ref, hp_ref, d0_ref, d1_ref, w_ref, b_ref,
                   out_ref):
    dis = _dis(d0_ref, d1_ref)
    t = dis * (s0_ref[...] + s1_ref[...] + hp_ref[...])
    h = lax.dot_general(t, w_ref[...], _CT,
                        preferred_element_type=jnp.float32) + b_ref[...]
    out_ref[...] = dis * jnp.maximum(h, 0.0)


def _tc_final_body(s0_ref, s1_ref, hp_ref, d0_ref, d1_ref, w_ref, b_ref,
                   wo_ref, bo_ref, out_ref):
    dis = _dis(d0_ref, d1_ref)
    t = dis * (s0_ref[...] + s1_ref[...] + hp_ref[...])
    h = lax.dot_general(t, w_ref[...], _CT,
                        preferred_element_type=jnp.float32) + b_ref[...]
    h = jnp.maximum(h, 0.0)
    out_ref[...] = lax.dot_general(h, wo_ref[...], _CT,
                                   preferred_element_type=jnp.float32) + bo_ref[...]


_deg_spec = pl.BlockSpec((R, 1), _row)
_feat_spec = pl.BlockSpec((R, H), _row)
_w_spec = pl.BlockSpec((H, H), _fix)
_b_spec = pl.BlockSpec((1, H), _fix)

_tc_prep = pl.pallas_call(
    _tc_prep_body,
    grid=(G,),
    in_specs=[pl.BlockSpec((R, D), _row), pl.BlockSpec((H, D), _fix),
              _b_spec, _deg_spec, _deg_spec],
    out_specs=_feat_spec,
    out_shape=jax.ShapeDtypeStruct((NPAD, H), jnp.float32),
)

_tc_layer = pl.pallas_call(
    _tc_layer_body,
    grid=(G,),
    in_specs=[_feat_spec, _feat_spec, _feat_spec, _deg_spec, _deg_spec,
              _w_spec, _b_spec],
    out_specs=_feat_spec,
    out_shape=jax.ShapeDtypeStruct((NPAD, H), jnp.float32),
)

_tc_final = pl.pallas_call(
    _tc_final_body,
    grid=(G,),
    in_specs=[_feat_spec, _feat_spec, _feat_spec, _deg_spec, _deg_spec,
              _w_spec, _b_spec, _w_spec, _b_spec],
    out_specs=_feat_spec,
    out_shape=jax.ShapeDtypeStruct((NPAD, H), jnp.float32),
)


@jax.jit
def kernel(x, edge_index, W_enc, b_enc, W1, b1, W2, b2, W_out, b_out):
    npw = EPW - E // NW
    srcw = edge_index[0].reshape(NW, E // NW)
    dstw = edge_index[1].reshape(NW, E // NW)
    src_i = jnp.concatenate(
        [srcw, jnp.zeros((NW, npw), jnp.int32)], axis=1).reshape(NW, NCHUNK, 1, K)
    dst_i = jnp.concatenate(
        [dstw, jnp.broadcast_to(N + jnp.arange(npw, dtype=jnp.int32), (NW, npw))],
        axis=1).reshape(NW, NCHUNK, 1, K)
    zeros_deg = jnp.zeros((NPAD,), jnp.float32)
    zeros_h = jnp.zeros((NPAD, H), jnp.float32)
    xp = jnp.zeros((NPAD, D), jnp.float32).at[:N].set(x)

    deg = _sc_degree(dst_i, zeros_deg)
    d0 = deg[:NPAD].reshape(NPAD, 1)
    d1 = deg[NPAD:].reshape(NPAD, 1)

    hp0 = _tc_prep(xp, W_enc, b_enc.reshape(1, H), d0, d1)
    s = _sc_scatter(hp0, src_i, dst_i, zeros_h)
    hp1 = _tc_layer(s[0], s[1], hp0, d0, d1, W1, b1.reshape(1, H))
    s2 = _sc_scatter(hp1, src_i, dst_i, zeros_h)

    wo = jnp.zeros((H, H), jnp.float32).at[:C].set(W_out)
    bo = jnp.zeros((1, H), jnp.float32).at[0, :C].set(b_out)
    out = _tc_final(s2[0], s2[1], hp1, d0, d1, W2, b2.reshape(1, H), wo, bo)
    return out[:N, :C]


# speedup vs baseline: 10.6443x; 1.0897x over previous
import functools

import jax
import jax.numpy as jnp
from jax import lax
from jax.experimental import pallas as pl
from jax.experimental.pallas import tpu as pltpu
from jax.experimental.pallas import tpu_sc as plsc

N = 10000
NPAD = 10240
E = 320000
D = 128
H = 128
C = 40

NC = 2
NS = 16
NW = NC * NS
EPW = 10240
EPAD = NW * EPW
K = 128
NCHUNK = EPW // K
RPT = NPAD // NS

_mesh = plsc.VectorSubcoreMesh(core_axis_name="c", subcore_axis_name="s")


@functools.partial(
    pl.kernel,
    out_type=jax.ShapeDtypeStruct((NC * NPAD,), jnp.float32),
    mesh=_mesh,
    scratch_types=[
        pltpu.VMEM_SHARED((NPAD,), jnp.float32),
        pltpu.VMEM((2, 1, K), jnp.int32),
        pltpu.VMEM((K,), jnp.float32),
    ],
)
def _sc_degree(dst_hbm, zeros_hbm, out_hbm, deg_sh, ring, ones_v):
    core = lax.axis_index("c")
    sub = lax.axis_index("s")
    wid = sub * NC + core
    row0 = sub * RPT
    for r in range(K // 16):
        ones_v[pl.ds(16 * r, 16)] = jnp.ones((16,), jnp.float32)
    pltpu.sync_copy(zeros_hbm.at[pl.ds(row0, RPT)], deg_sh.at[pl.ds(row0, RPT)])
    plsc.subcore_barrier()

    pltpu.sync_copy(dst_hbm.at[wid, 0], ring.at[0])

    @pl.loop(0, NCHUNK, step=2)
    def _(j):
        for b in range(2):
            c = j + b

            @pl.when(c + 1 < NCHUNK)
            def _():
                pltpu.sync_copy(dst_hbm.at[wid, c + 1], ring.at[1 - b])

            pltpu.sync_copy(ones_v, deg_sh.at[ring.at[b, 0]], add=True)

    plsc.subcore_barrier()
    pltpu.sync_copy(deg_sh.at[pl.ds(row0, RPT)],
                    out_hbm.at[pl.ds(core * NPAD + row0, RPT)])


@functools.partial(
    pl.kernel,
    out_type=jax.ShapeDtypeStruct((NC, NPAD, H), jnp.float32),
    mesh=_mesh,
    scratch_types=[
        pltpu.VMEM_SHARED((NPAD, H), jnp.float32),
        pltpu.VMEM((NCHUNK, 1, K), jnp.int32),
        pltpu.VMEM((2, 1, K), jnp.int32),
        pltpu.VMEM((2, K, H), jnp.float32),
        pltpu.SemaphoreType.DMA,
        pltpu.SemaphoreType.DMA,
        pltpu.SemaphoreType.DMA,
        pltpu.SemaphoreType.DMA,
    ],
)
def _sc_scatter(hp_hbm, src_hbm, dst_hbm, zeros_hbm, out_hbm,
                agg_sh, src_v, ring_d, rows_v, gsem0, gsem1, dsem0, dsem1):
    core = lax.axis_index("c")
    sub = lax.axis_index("s")
    wid = sub * NC + core
    row0 = sub * RPT
    pltpu.sync_copy(zeros_hbm.at[pl.ds(row0, RPT)], agg_sh.at[pl.ds(row0, RPT)])
    pltpu.sync_copy(src_hbm.at[wid], src_v)
    plsc.subcore_barrier()

    pltpu.async_copy(dst_hbm.at[wid, 0], ring_d.at[0], dsem0)
    pltpu.async_copy(hp_hbm.at[src_v.at[0, 0]], rows_v.at[0], gsem0)

    @pl.loop(0, NCHUNK, step=2)
    def _(j):
        for b in range(2):
            c = j + b
            gsem_b = gsem0 if b == 0 else gsem1
            gsem_o = gsem1 if b == 0 else gsem0
            dsem_b = dsem0 if b == 0 else dsem1
            dsem_o = dsem1 if b == 0 else dsem0

            @pl.when(c + 1 < NCHUNK)
            def _():
                pltpu.async_copy(dst_hbm.at[wid, c + 1], ring_d.at[1 - b],
                                 dsem_o)
                pltpu.async_copy(hp_hbm.at[src_v.at[c + 1, 0]],
                                 rows_v.at[1 - b], gsem_o)

            pltpu.make_async_copy(hp_hbm.at[src_v.at[c, 0]],
                                  rows_v.at[b], gsem_b).wait()
            pltpu.make_async_copy(dst_hbm.at[wid, c], ring_d.at[b],
                                  dsem_b).wait()
            pltpu.sync_copy(rows_v.at[b], agg_sh.at[ring_d.at[b, 0]], add=True)

    plsc.subcore_barrier()
    pltpu.sync_copy(agg_sh.at[pl.ds(row0, RPT)],
                    out_hbm.at[core, pl.ds(row0, RPT)])


R = 1024
G = NPAD // R

_row = lambda i: (i, 0)
_fix = lambda i: (0, 0)
_CT = (((1,), (1,)), ((), ()))


def _dis(d0_ref, d1_ref):
    return lax.rsqrt(1.0 + d0_ref[...] + d1_ref[...])


def _tc_prep_body(x_ref, w_ref, b_ref, d0_ref, d1_ref, out_ref):
    h = lax.dot_general(x_ref[...], w_ref[...], _CT,
                        preferred_element_type=jnp.float32) + b_ref[...]
    out_ref[...] = _dis(d0_ref, d1_ref) * h


def _tc_layer_body(s0_ref, s1_ref, hp_ref, d0_ref, d1_ref, w_ref, b_ref,
                   out_ref):
    dis = _dis(d0_ref, d1_ref)
    t = dis * (s0_ref[...] + s1_ref[...] + hp_ref[...])
    h = lax.dot_general(t, w_ref[...], _CT,
                        preferred_element_type=jnp.float32) + b_ref[...]
    out_ref[...] = dis * jnp.maximum(h, 0.0)


def _tc_final_body(s0_ref, s1_ref, hp_ref, d0_ref, d1_ref, w_ref, b_ref,
                   wo_ref, bo_ref, out_ref):
    dis = _dis(d0_ref, d1_ref)
    t = dis * (s0_ref[...] + s1_ref[...] + hp_ref[...])
    h = lax.dot_general(t, w_ref[...], _CT,
                        preferred_element_type=jnp.float32) + b_ref[...]
    h = jnp.maximum(h, 0.0)
    out_ref[...] = lax.dot_general(h, wo_ref[...], _CT,
                                   preferred_element_type=jnp.float32) + bo_ref[...]


_deg_spec = pl.BlockSpec((R, 1), _row)
_feat_spec = pl.BlockSpec((R, H), _row)
_w_spec = pl.BlockSpec((H, H), _fix)
_b_spec = pl.BlockSpec((1, H), _fix)

_tc_prep = pl.pallas_call(
    _tc_prep_body,
    grid=(G,),
    in_specs=[pl.BlockSpec((R, D), _row), pl.BlockSpec((H, D), _fix),
              _b_spec, _deg_spec, _deg_spec],
    out_specs=_feat_spec,
    out_shape=jax.ShapeDtypeStruct((NPAD, H), jnp.float32),
)

_tc_layer = pl.pallas_call(
    _tc_layer_body,
    grid=(G,),
    in_specs=[_feat_spec, _feat_spec, _feat_spec, _deg_spec, _deg_spec,
              _w_spec, _b_spec],
    out_specs=_feat_spec,
    out_shape=jax.ShapeDtypeStruct((NPAD, H), jnp.float32),
)

_tc_final = pl.pallas_call(
    _tc_final_body,
    grid=(G,),
    in_specs=[_feat_spec, _feat_spec, _feat_spec, _deg_spec, _deg_spec,
              _w_spec, _b_spec, _w_spec, _b_spec],
    out_specs=_feat_spec,
    out_shape=jax.ShapeDtypeStruct((NPAD, H), jnp.float32),
)


@jax.jit
def kernel(x, edge_index, W_enc, b_enc, W1, b1, W2, b2, W_out, b_out):
    npw = EPW - E // NW
    srcw = edge_index[0].reshape(NW, E // NW)
    dstw = edge_index[1].reshape(NW, E // NW)
    src_i = jnp.concatenate(
        [srcw, jnp.zeros((NW, npw), jnp.int32)], axis=1).reshape(NW, NCHUNK, 1, K)
    dst_i = jnp.concatenate(
        [dstw, jnp.broadcast_to(N + jnp.arange(npw, dtype=jnp.int32), (NW, npw))],
        axis=1).reshape(NW, NCHUNK, 1, K)
    zeros_deg = jnp.zeros((NPAD,), jnp.float32)
    zeros_h = jnp.zeros((NPAD, H), jnp.float32)
    xp = jnp.zeros((NPAD, D), jnp.float32).at[:N].set(x)

    deg = _sc_degree(dst_i, zeros_deg)
    d0 = deg[:NPAD].reshape(NPAD, 1)
    d1 = deg[NPAD:].reshape(NPAD, 1)

    hp0 = _tc_prep(xp, W_enc, b_enc.reshape(1, H), d0, d1)
    s = _sc_scatter(hp0, src_i, dst_i, zeros_h)
    hp1 = _tc_layer(s[0], s[1], hp0, d0, d1, W1, b1.reshape(1, H))
    s2 = _sc_scatter(hp1, src_i, dst_i, zeros_h)

    wo = jnp.zeros((H, H), jnp.float32).at[:C].set(W_out)
    bo = jnp.zeros((1, H), jnp.float32).at[0, :C].set(b_out)
    out = _tc_final(s2[0], s2[1], hp1, d0, d1, W2, b2.reshape(1, H), wo, bo)
    return out[:N, :C]
